# baseline (speedup 1.0000x reference)
import functools

import jax
import jax.numpy as jnp
from jax import lax
from jax.experimental import pallas as pl
from jax.experimental.pallas import tpu as pltpu
from jax.experimental.pallas import tpu_sc as plsc

N = 10000
E = 320000
D = 128
NC = 2
NS = 16
NW = NC * NS
K = 64
CHUNKS = 160
EPT = K * CHUNKS
E_PAD = NW * EPT
N_PAD = 10240
RPT = N_PAD // NS
WB = RPT // K
DEGW = 16


def _make_sc_agg():
  out_type = jax.ShapeDtypeStruct((NC, N_PAD, D), jnp.float32)
  scratch = [
      pltpu.VMEM((K,), jnp.int32),
      pltpu.VMEM((K,), jnp.int32),
      pltpu.VMEM((K,), jnp.int32),
      pltpu.VMEM((K,), jnp.int32),
      pltpu.VMEM((K, D), jnp.float32),
      pltpu.VMEM((K, D), jnp.float32),
      pltpu.VMEM_SHARED((N_PAD, D), jnp.float32),
      pltpu.SemaphoreType.DMA,
      pltpu.SemaphoreType.DMA,
      pltpu.SemaphoreType.DMA,
      pltpu.SemaphoreType.DMA,
      pltpu.SemaphoreType.DMA,
      pltpu.SemaphoreType.DMA,
  ]
  mesh = plsc.VectorSubcoreMesh(core_axis_name="c", subcore_axis_name="s")

  def body(h_hbm, src_hbm, dst_hbm, znd_hbm,
           agg_out, src0, dst0, src1, dst1, rows0, rows1, agg_sh,
           semS0, semD0, semS1, semD1, semG0, semG1):
    c = lax.axis_index("c")
    s = lax.axis_index("s")
    wid = s * NC + c
    r0 = pl.multiple_of(s * RPT, 8)
    pltpu.sync_copy(znd_hbm.at[pl.ds(0, K)], rows0)

    def zinit(j, carry):
      pltpu.sync_copy(rows0, agg_sh.at[pl.ds(r0 + j * K, K)])
      return carry

    lax.fori_loop(0, WB, zinit, 0)
    plsc.subcore_barrier()

    ebase = wid * EPT

    def issue_idx(i, srcb, dstb, semS, semD):
      base = pl.multiple_of(ebase + i * K, 8)
      pltpu.async_copy(src_hbm.at[pl.ds(base, K)], srcb, semS)
      pltpu.async_copy(dst_hbm.at[pl.ds(base, K)], dstb, semD)

    def wait_idx(srcb, dstb, semS, semD):
      pltpu.make_async_copy(src_hbm.at[pl.ds(0, K)], srcb, semS).wait()
      pltpu.make_async_copy(dst_hbm.at[pl.ds(0, K)], dstb, semD).wait()

    def issue_gather(srcb, rowsb, semG):
      pltpu.async_copy(h_hbm.at[srcb], rowsb, semG)

    def wait_gather(rowsb, semG):
      pltpu.make_async_copy(h_hbm.at[pl.ds(0, K)], rowsb, semG).wait()

    def scatter(rowsb, dstb):
      pltpu.sync_copy(rowsb, agg_sh.at[dstb], add=True)

    issue_idx(0, src0, dst0, semS0, semD0)
    issue_idx(1, src1, dst1, semS1, semD1)
    wait_idx(src0, dst0, semS0, semD0)
    issue_gather(src0, rows0, semG0)

    def pair(p, carry):
      i = 2 * p
      wait_idx(src1, dst1, semS1, semD1)
      wait_gather(rows0, semG0)
      issue_gather(src1, rows1, semG1)
      scatter(rows0, dst0)
      issue_idx(i + 2, src0, dst0, semS0, semD0)
      wait_gather(rows1, semG1)
      scatter(rows1, dst1)
      wait_idx(src0, dst0, semS0, semD0)
      issue_gather(src0, rows0, semG0)
      issue_idx(i + 3, src1, dst1, semS1, semD1)
      return carry

    lax.fori_loop(0, (CHUNKS - 2) // 2, pair, 0)
    wait_idx(src1, dst1, semS1, semD1)
    wait_gather(rows0, semG0)
    issue_gather(src1, rows1, semG1)
    scatter(rows0, dst0)
    wait_gather(rows1, semG1)
    scatter(rows1, dst1)
    plsc.subcore_barrier()

    def wback(j, carry):
      pltpu.sync_copy(agg_sh.at[pl.ds(r0 + j * K, K)], rows0)
      pltpu.sync_copy(rows0, agg_out.at[c, pl.ds(r0 + j * K, K)])
      return carry

    lax.fori_loop(0, WB, wback, 0)

  return pl.kernel(body, out_type=out_type, mesh=mesh, scratch_types=scratch,
                   compiler_params=pltpu.CompilerParams(use_tc_tiling_on_sc=False))


def _make_sc_deg():
  out_type = jax.ShapeDtypeStruct((NC, N_PAD, DEGW), jnp.float32)
  scratch = [
      pltpu.VMEM((K,), jnp.int32),
      pltpu.VMEM((K,), jnp.int32),
      pltpu.VMEM((K, DEGW), jnp.float32),
      pltpu.VMEM((RPT, DEGW), jnp.float32),
      pltpu.VMEM_SHARED((N_PAD, DEGW), jnp.float32),
      pltpu.SemaphoreType.DMA,
      pltpu.SemaphoreType.DMA,
  ]
  mesh = plsc.VectorSubcoreMesh(core_axis_name="c", subcore_axis_name="s")

  def body(dst_hbm, z16_hbm, ones_hbm, deg_out, dst0, dst1, ones_v, dbuf,
           deg_sh, semD0, semD1):
    c = lax.axis_index("c")
    s = lax.axis_index("s")
    wid = s * NC + c
    r0 = pl.multiple_of(s * RPT, 8)
    pltpu.sync_copy(z16_hbm.at[pl.ds(0, RPT)], dbuf)
    pltpu.sync_copy(dbuf, deg_sh.at[pl.ds(r0, RPT)])
    pltpu.sync_copy(ones_hbm, ones_v)
    plsc.subcore_barrier()

    ebase = wid * EPT

    def issue(i, dstb, semD):
      base = pl.multiple_of(ebase + i * K, 8)
      pltpu.async_copy(dst_hbm.at[pl.ds(base, K)], dstb, semD)

    def wait(dstb, semD):
      pltpu.make_async_copy(dst_hbm.at[pl.ds(0, K)], dstb, semD).wait()

    issue(0, dst0, semD0)
    issue(1, dst1, semD1)

    def pair(p, carry):
      i = 2 * p
      wait(dst0, semD0)
      pltpu.sync_copy(ones_v, deg_sh.at[dst0], add=True)
      issue(i + 2, dst0, semD0)
      wait(dst1, semD1)
      pltpu.sync_copy(ones_v, deg_sh.at[dst1], add=True)
      issue(i + 3, dst1, semD1)
      return carry

    lax.fori_loop(0, (CHUNKS - 2) // 2, pair, 0)
    wait(dst0, semD0)
    pltpu.sync_copy(ones_v, deg_sh.at[dst0], add=True)
    wait(dst1, semD1)
    pltpu.sync_copy(ones_v, deg_sh.at[dst1], add=True)
    plsc.subcore_barrier()
    pltpu.sync_copy(deg_sh.at[pl.ds(r0, RPT)], dbuf)
    pltpu.sync_copy(dbuf, deg_out.at[c, pl.ds(r0, RPT)])

  return pl.kernel(body, out_type=out_type, mesh=mesh, scratch_types=scratch,
                   compiler_params=pltpu.CompilerParams(use_tc_tiling_on_sc=False))


def _tc_body(relu, h_ref, a0_ref, a1_ref, d0_ref, d1_ref, ws_ref, wn_ref,
             b_ref, g_ref, bt_ref, o_ref):
  h = h_ref[...]
  agg = a0_ref[...] + a1_ref[...]
  deg = d0_ref[:, 0:1] + d1_ref[:, 0:1]
  h_neigh = agg * (1.0 / jnp.maximum(deg, 1.0))
  out = (jnp.dot(h, ws_ref[...], preferred_element_type=jnp.float32)
         + jnp.dot(h_neigh, wn_ref[...], preferred_element_type=jnp.float32)
         + b_ref[...])
  if relu:
    out = jnp.maximum(out, 0.0)
  mu = jnp.mean(out, axis=-1, keepdims=True)
  var = jnp.mean((out - mu) ** 2, axis=-1, keepdims=True)
  o_ref[...] = (out - mu) * lax.rsqrt(var + 1e-5) * g_ref[...] + bt_ref[...]


BN = 1000


def _tc_dense(h, a0, a1, d0, d1, ws, wn, b, g, bt, relu):
  bs_rows = pl.BlockSpec((BN, D), lambda i: (i, 0))
  bs_deg = pl.BlockSpec((BN, DEGW), lambda i: (i, 0))
  bs_w = pl.BlockSpec((D, D), lambda i: (0, 0))
  bs_v = pl.BlockSpec((1, D), lambda i: (0, 0))
  return pl.pallas_call(
      functools.partial(_tc_body, relu),
      grid=(N // BN,),
      in_specs=[bs_rows, bs_rows, bs_rows, bs_deg, bs_deg,
                bs_w, bs_w, bs_v, bs_v, bs_v],
      out_specs=bs_rows,
      out_shape=jax.ShapeDtypeStruct((N, D), jnp.float32),
  )(h, a0, a1, d0, d1, ws, wn, b, g, bt)


_sc_agg = _make_sc_agg()
_sc_deg = _make_sc_deg()


def kernel(x, edge_index, W_self0, W_neigh0, b0, W_self1, W_neigh1, b1,
           ln_gamma, ln_beta):
  src = jnp.concatenate([edge_index[0].astype(jnp.int32),
                         jnp.zeros((E_PAD - E,), jnp.int32)])
  dst = jnp.concatenate([edge_index[1].astype(jnp.int32),
                         N + jnp.arange(E_PAD - E, dtype=jnp.int32)
                         % (N_PAD - N)])
  znd = jnp.zeros((N_PAD, D), jnp.float32)
  z16 = jnp.zeros((N_PAD, DEGW), jnp.float32)
  ones = jnp.ones((K, DEGW), jnp.float32)
  b0r, b1r = b0.reshape(1, D), b1.reshape(1, D)
  gr, btr = ln_gamma.reshape(1, D), ln_beta.reshape(1, D)

  degp = _sc_deg(dst, z16, ones)
  agg0 = _sc_agg(x, src, dst, znd)
  h1 = _tc_dense(x, agg0[0], agg0[1], degp[0], degp[1],
                 W_self0, W_neigh0, b0r, gr, btr, True)
  agg1 = _sc_agg(h1, src, dst, znd)
  out = _tc_dense(h1, agg1[0], agg1[1], degp[0], degp[1],
                  W_self1, W_neigh1, b1r, gr, btr, False)
  return out


# speedup vs baseline: 1.7344x; 1.7344x over previous
import functools

import jax
import jax.numpy as jnp
from jax import lax
from jax.experimental import pallas as pl
from jax.experimental.pallas import tpu as pltpu
from jax.experimental.pallas import tpu_sc as plsc

N = 10000
E = 320000
D = 128
NC = 2
NS = 16
NW = NC * NS
K = 80
CHUNKS = 126
EPT = K * CHUNKS
E_PAD = NW * EPT
N_PAD = 10240
RPT = N_PAD // NS
WB = RPT // K
DEGW = 16


def _make_sc_agg():
  out_type = jax.ShapeDtypeStruct((NC, N_PAD, D), jnp.float32)
  scratch = [
      pltpu.VMEM((K,), jnp.int32),
      pltpu.VMEM((K,), jnp.int32),
      pltpu.VMEM((K,), jnp.int32),
      pltpu.VMEM((K,), jnp.int32),
      pltpu.VMEM((K, D), jnp.float32),
      pltpu.VMEM((K, D), jnp.float32),
      pltpu.VMEM_SHARED((N_PAD, D), jnp.float32),
      pltpu.SemaphoreType.DMA,
      pltpu.SemaphoreType.DMA,
      pltpu.SemaphoreType.DMA,
      pltpu.SemaphoreType.DMA,
      pltpu.SemaphoreType.DMA,
      pltpu.SemaphoreType.DMA,
  ]
  mesh = plsc.VectorSubcoreMesh(core_axis_name="c", subcore_axis_name="s")

  def body(h_hbm, src_hbm, dst_hbm, znd_hbm,
           agg_out, src0, dst0, src1, dst1, rows0, rows1, agg_sh,
           semS0, semD0, semS1, semD1, semG0, semG1):
    c = lax.axis_index("c")
    s = lax.axis_index("s")
    wid = s * NC + c
    r0 = pl.multiple_of(s * RPT, 8)
    pltpu.sync_copy(znd_hbm.at[pl.ds(0, K)], rows0)

    def zinit(j, carry):
      pltpu.sync_copy(rows0, agg_sh.at[pl.ds(r0 + j * K, K)])
      return carry

    lax.fori_loop(0, WB, zinit, 0)
    plsc.subcore_barrier()

    ebase = wid * EPT

    def issue_idx(i, srcb, dstb, semS, semD):
      base = pl.multiple_of(ebase + i * K, 8)
      pltpu.async_copy(src_hbm.at[pl.ds(base, K)], srcb, semS)
      pltpu.async_copy(dst_hbm.at[pl.ds(base, K)], dstb, semD)

    def wait_idx(srcb, dstb, semS, semD):
      pltpu.make_async_copy(src_hbm.at[pl.ds(0, K)], srcb, semS).wait()
      pltpu.make_async_copy(dst_hbm.at[pl.ds(0, K)], dstb, semD).wait()

    def issue_gather(srcb, rowsb, semG):
      pltpu.async_copy(h_hbm.at[srcb], rowsb, semG)

    def wait_gather(rowsb, semG):
      pltpu.make_async_copy(h_hbm.at[pl.ds(0, K)], rowsb, semG).wait()

    def scatter(rowsb, dstb):
      pltpu.sync_copy(rowsb, agg_sh.at[dstb], add=True)

    issue_idx(0, src0, dst0, semS0, semD0)
    issue_idx(1, src1, dst1, semS1, semD1)
    wait_idx(src0, dst0, semS0, semD0)
    issue_gather(src0, rows0, semG0)

    def pair(p, carry):
      i = 2 * p
      wait_idx(src1, dst1, semS1, semD1)
      wait_gather(rows0, semG0)
      issue_gather(src1, rows1, semG1)
      scatter(rows0, dst0)
      issue_idx(i + 2, src0, dst0, semS0, semD0)
      wait_gather(rows1, semG1)
      scatter(rows1, dst1)
      wait_idx(src0, dst0, semS0, semD0)
      issue_gather(src0, rows0, semG0)
      issue_idx(i + 3, src1, dst1, semS1, semD1)
      return carry

    lax.fori_loop(0, (CHUNKS - 2) // 2, pair, 0)
    wait_idx(src1, dst1, semS1, semD1)
    wait_gather(rows0, semG0)
    issue_gather(src1, rows1, semG1)
    scatter(rows0, dst0)
    wait_gather(rows1, semG1)
    scatter(rows1, dst1)
    plsc.subcore_barrier()

    def wback(j, carry):
      pltpu.sync_copy(agg_sh.at[pl.ds(r0 + j * K, K)], rows0)
      pltpu.sync_copy(rows0, agg_out.at[c, pl.ds(r0 + j * K, K)])
      return carry

    lax.fori_loop(0, WB, wback, 0)

  return pl.kernel(body, out_type=out_type, mesh=mesh, scratch_types=scratch,
                   compiler_params=pltpu.CompilerParams(use_tc_tiling_on_sc=False))


def _make_sc_deg():
  out_type = jax.ShapeDtypeStruct((NC, N_PAD, DEGW), jnp.float32)
  scratch = [
      pltpu.VMEM((K,), jnp.int32),
      pltpu.VMEM((K,), jnp.int32),
      pltpu.VMEM((K, DEGW), jnp.float32),
      pltpu.VMEM((RPT, DEGW), jnp.float32),
      pltpu.VMEM_SHARED((N_PAD, DEGW), jnp.float32),
      pltpu.SemaphoreType.DMA,
      pltpu.SemaphoreType.DMA,
  ]
  mesh = plsc.VectorSubcoreMesh(core_axis_name="c", subcore_axis_name="s")

  def body(dst_hbm, z16_hbm, ones_hbm, deg_out, dst0, dst1, ones_v, dbuf,
           deg_sh, semD0, semD1):
    c = lax.axis_index("c")
    s = lax.axis_index("s")
    wid = s * NC + c
    r0 = pl.multiple_of(s * RPT, 8)
    pltpu.sync_copy(z16_hbm.at[pl.ds(0, RPT)], dbuf)
    pltpu.sync_copy(dbuf, deg_sh.at[pl.ds(r0, RPT)])
    pltpu.sync_copy(ones_hbm, ones_v)
    plsc.subcore_barrier()

    ebase = wid * EPT

    def issue(i, dstb, semD):
      base = pl.multiple_of(ebase + i * K, 8)
      pltpu.async_copy(dst_hbm.at[pl.ds(base, K)], dstb, semD)

    def wait(dstb, semD):
      pltpu.make_async_copy(dst_hbm.at[pl.ds(0, K)], dstb, semD).wait()

    issue(0, dst0, semD0)
    issue(1, dst1, semD1)

    def pair(p, carry):
      i = 2 * p
      wait(dst0, semD0)
      pltpu.sync_copy(ones_v, deg_sh.at[dst0], add=True)
      issue(i + 2, dst0, semD0)
      wait(dst1, semD1)
      pltpu.sync_copy(ones_v, deg_sh.at[dst1], add=True)
      issue(i + 3, dst1, semD1)
      return carry

    lax.fori_loop(0, (CHUNKS - 2) // 2, pair, 0)
    wait(dst0, semD0)
    pltpu.sync_copy(ones_v, deg_sh.at[dst0], add=True)
    wait(dst1, semD1)
    pltpu.sync_copy(ones_v, deg_sh.at[dst1], add=True)
    plsc.subcore_barrier()
    pltpu.sync_copy(deg_sh.at[pl.ds(r0, RPT)], dbuf)
    pltpu.sync_copy(dbuf, deg_out.at[c, pl.ds(r0, RPT)])

  return pl.kernel(body, out_type=out_type, mesh=mesh, scratch_types=scratch,
                   compiler_params=pltpu.CompilerParams(use_tc_tiling_on_sc=False))


def _tc_body(relu, h_ref, a0_ref, a1_ref, d0_ref, d1_ref, ws_ref, wn_ref,
             b_ref, g_ref, bt_ref, o_ref):
  h = h_ref[...]
  agg = a0_ref[...] + a1_ref[...]
  deg = d0_ref[:, 0:1] + d1_ref[:, 0:1]
  h_neigh = agg * (1.0 / jnp.maximum(deg, 1.0))
  out = (jnp.dot(h, ws_ref[...], preferred_element_type=jnp.float32)
         + jnp.dot(h_neigh, wn_ref[...], preferred_element_type=jnp.float32)
         + b_ref[...])
  if relu:
    out = jnp.maximum(out, 0.0)
  mu = jnp.mean(out, axis=-1, keepdims=True)
  var = jnp.mean((out - mu) ** 2, axis=-1, keepdims=True)
  o_ref[...] = (out - mu) * lax.rsqrt(var + 1e-5) * g_ref[...] + bt_ref[...]


BN = 1000


def _tc_dense(h, a0, a1, d0, d1, ws, wn, b, g, bt, relu):
  bs_rows = pl.BlockSpec((BN, D), lambda i: (i, 0))
  bs_deg = pl.BlockSpec((BN, DEGW), lambda i: (i, 0))
  bs_w = pl.BlockSpec((D, D), lambda i: (0, 0))
  bs_v = pl.BlockSpec((1, D), lambda i: (0, 0))
  return pl.pallas_call(
      functools.partial(_tc_body, relu),
      grid=(N // BN,),
      in_specs=[bs_rows, bs_rows, bs_rows, bs_deg, bs_deg,
                bs_w, bs_w, bs_v, bs_v, bs_v],
      out_specs=bs_rows,
      out_shape=jax.ShapeDtypeStruct((N, D), jnp.float32),
  )(h, a0, a1, d0, d1, ws, wn, b, g, bt)


_sc_agg = _make_sc_agg()
_sc_deg = _make_sc_deg()


def kernel(x, edge_index, W_self0, W_neigh0, b0, W_self1, W_neigh1, b1,
           ln_gamma, ln_beta):
  src = jnp.concatenate([edge_index[0].astype(jnp.int32),
                         jnp.zeros((E_PAD - E,), jnp.int32)])
  dst = jnp.concatenate([edge_index[1].astype(jnp.int32),
                         N + jnp.arange(E_PAD - E, dtype=jnp.int32)
                         % (N_PAD - N)])
  znd = jnp.zeros((N_PAD, D), jnp.float32)
  z16 = jnp.zeros((N_PAD, DEGW), jnp.float32)
  ones = jnp.ones((K, DEGW), jnp.float32)
  b0r, b1r = b0.reshape(1, D), b1.reshape(1, D)
  gr, btr = ln_gamma.reshape(1, D), ln_beta.reshape(1, D)

  degp = _sc_deg(dst, z16, ones)
  agg0 = _sc_agg(x, src, dst, znd)
  h1 = _tc_dense(x, agg0[0], agg0[1], degp[0], degp[1],
                 W_self0, W_neigh0, b0r, gr, btr, True)
  agg1 = _sc_agg(h1, src, dst, znd)
  out = _tc_dense(h1, agg1[0], agg1[1], degp[0], degp[1],
                  W_self1, W_neigh1, b1r, gr, btr, False)
  return out


# speedup vs baseline: 1.9966x; 1.1512x over previous
import functools

import jax
import jax.numpy as jnp
from jax import lax
from jax.experimental import pallas as pl
from jax.experimental.pallas import tpu as pltpu
from jax.experimental.pallas import tpu_sc as plsc

N = 10000
E = 320000
D = 128
NC = 2
NS = 16
NW = NC * NS
K = 80
CHUNKS = 126
EPT = K * CHUNKS
E_PAD = NW * EPT
N_PAD = 10240
RPT = N_PAD // NS
WB = RPT // K
DEGW = 16


def _make_sc_agg():
  out_type = jax.ShapeDtypeStruct((NC, N_PAD, D), jnp.float32)
  scratch = ([pltpu.VMEM((K,), jnp.int32) for _ in range(3)]
             + [pltpu.VMEM((K,), jnp.int32) for _ in range(3)]
             + [pltpu.VMEM((K, D), jnp.float32) for _ in range(3)]
             + [pltpu.VMEM_SHARED((N_PAD, D), jnp.float32)]
             + [pltpu.SemaphoreType.DMA for _ in range(9)])
  mesh = plsc.VectorSubcoreMesh(core_axis_name="c", subcore_axis_name="s")

  def body(h_hbm, src_hbm, dst_hbm, znd_hbm, agg_out,
           srcA, srcB, srcC, dstA, dstB, dstC, rowsA, rowsB, rowsC, agg_sh,
           sSA, sSB, sSC, sDA, sDB, sDC, sGA, sGB, sGC):
    srcs, dsts, rows = (srcA, srcB, srcC), (dstA, dstB, dstC), (rowsA, rowsB, rowsC)
    semS, semD, semG = (sSA, sSB, sSC), (sDA, sDB, sDC), (sGA, sGB, sGC)
    rows0 = rowsA
    c = lax.axis_index("c")
    s = lax.axis_index("s")
    wid = s * NC + c
    r0 = pl.multiple_of(s * RPT, 8)
    pltpu.sync_copy(znd_hbm.at[pl.ds(0, K)], rows0)

    def zinit(j, carry):
      pltpu.sync_copy(rows0, agg_sh.at[pl.ds(r0 + j * K, K)])
      return carry

    lax.fori_loop(0, WB, zinit, 0)
    plsc.subcore_barrier()

    ebase = wid * EPT

    def issue_idx(i, b):
      base = pl.multiple_of(ebase + i * K, 8)
      pltpu.async_copy(src_hbm.at[pl.ds(base, K)], srcs[b], semS[b])
      pltpu.async_copy(dst_hbm.at[pl.ds(base, K)], dsts[b], semD[b])

    def wait_idx(b):
      pltpu.make_async_copy(src_hbm.at[pl.ds(0, K)], srcs[b], semS[b]).wait()
      pltpu.make_async_copy(dst_hbm.at[pl.ds(0, K)], dsts[b], semD[b]).wait()

    def issue_gather(b):
      pltpu.async_copy(h_hbm.at[srcs[b]], rows[b], semG[b])

    def wait_gather(b):
      pltpu.make_async_copy(h_hbm.at[pl.ds(0, K)], rows[b], semG[b]).wait()

    def scatter(b):
      pltpu.sync_copy(rows[b], agg_sh.at[dsts[b]], add=True)

    issue_idx(0, 0)
    issue_idx(1, 1)
    issue_idx(2, 2)
    wait_idx(0)
    issue_gather(0)
    wait_idx(1)
    issue_gather(1)

    def triple(p, carry):
      i = 3 * p
      for b in range(3):
        b2 = (b + 2) % 3
        wait_idx(b2)
        issue_gather(b2)
        wait_gather(b)
        scatter(b)
        issue_idx(i + b + 3, b)
      return carry

    lax.fori_loop(0, (CHUNKS - 3) // 3, triple, 0)
    wait_idx(2)
    issue_gather(2)
    wait_gather(0)
    scatter(0)
    wait_gather(1)
    scatter(1)
    wait_gather(2)
    scatter(2)
    plsc.subcore_barrier()

    def wback(j, carry):
      pltpu.sync_copy(agg_sh.at[pl.ds(r0 + j * K, K)], rows0)
      pltpu.sync_copy(rows0, agg_out.at[c, pl.ds(r0 + j * K, K)])
      return carry

    lax.fori_loop(0, WB, wback, 0)

  return pl.kernel(body, out_type=out_type, mesh=mesh, scratch_types=scratch,
                   compiler_params=pltpu.CompilerParams(use_tc_tiling_on_sc=False))


def _make_sc_deg():
  out_type = jax.ShapeDtypeStruct((NC, N_PAD, DEGW), jnp.float32)
  scratch = [
      pltpu.VMEM((K,), jnp.int32),
      pltpu.VMEM((K,), jnp.int32),
      pltpu.VMEM((K, DEGW), jnp.float32),
      pltpu.VMEM((RPT, DEGW), jnp.float32),
      pltpu.VMEM_SHARED((N_PAD, DEGW), jnp.float32),
      pltpu.SemaphoreType.DMA,
      pltpu.SemaphoreType.DMA,
  ]
  mesh = plsc.VectorSubcoreMesh(core_axis_name="c", subcore_axis_name="s")

  def body(dst_hbm, z16_hbm, ones_hbm, deg_out, dst0, dst1, ones_v, dbuf,
           deg_sh, semD0, semD1):
    c = lax.axis_index("c")
    s = lax.axis_index("s")
    wid = s * NC + c
    r0 = pl.multiple_of(s * RPT, 8)
    pltpu.sync_copy(z16_hbm.at[pl.ds(0, RPT)], dbuf)
    pltpu.sync_copy(dbuf, deg_sh.at[pl.ds(r0, RPT)])
    pltpu.sync_copy(ones_hbm, ones_v)
    plsc.subcore_barrier()

    ebase = wid * EPT

    def issue(i, dstb, semD):
      base = pl.multiple_of(ebase + i * K, 8)
      pltpu.async_copy(dst_hbm.at[pl.ds(base, K)], dstb, semD)

    def wait(dstb, semD):
      pltpu.make_async_copy(dst_hbm.at[pl.ds(0, K)], dstb, semD).wait()

    issue(0, dst0, semD0)
    issue(1, dst1, semD1)

    def pair(p, carry):
      i = 2 * p
      wait(dst0, semD0)
      pltpu.sync_copy(ones_v, deg_sh.at[dst0], add=True)
      issue(i + 2, dst0, semD0)
      wait(dst1, semD1)
      pltpu.sync_copy(ones_v, deg_sh.at[dst1], add=True)
      issue(i + 3, dst1, semD1)
      return carry

    lax.fori_loop(0, (CHUNKS - 2) // 2, pair, 0)
    wait(dst0, semD0)
    pltpu.sync_copy(ones_v, deg_sh.at[dst0], add=True)
    wait(dst1, semD1)
    pltpu.sync_copy(ones_v, deg_sh.at[dst1], add=True)
    plsc.subcore_barrier()
    pltpu.sync_copy(deg_sh.at[pl.ds(r0, RPT)], dbuf)
    pltpu.sync_copy(dbuf, deg_out.at[c, pl.ds(r0, RPT)])

  return pl.kernel(body, out_type=out_type, mesh=mesh, scratch_types=scratch,
                   compiler_params=pltpu.CompilerParams(use_tc_tiling_on_sc=False))


def _tc_body(relu, h_ref, a0_ref, a1_ref, d0_ref, d1_ref, ws_ref, wn_ref,
             b_ref, g_ref, bt_ref, o_ref):
  h = h_ref[...]
  agg = a0_ref[...] + a1_ref[...]
  deg = d0_ref[:, 0:1] + d1_ref[:, 0:1]
  h_neigh = agg * (1.0 / jnp.maximum(deg, 1.0))
  out = (jnp.dot(h, ws_ref[...], preferred_element_type=jnp.float32)
         + jnp.dot(h_neigh, wn_ref[...], preferred_element_type=jnp.float32)
         + b_ref[...])
  if relu:
    out = jnp.maximum(out, 0.0)
  mu = jnp.mean(out, axis=-1, keepdims=True)
  var = jnp.mean((out - mu) ** 2, axis=-1, keepdims=True)
  o_ref[...] = (out - mu) * lax.rsqrt(var + 1e-5) * g_ref[...] + bt_ref[...]


BN = 1000


def _tc_dense(h, a0, a1, d0, d1, ws, wn, b, g, bt, relu):
  bs_rows = pl.BlockSpec((BN, D), lambda i: (i, 0))
  bs_deg = pl.BlockSpec((BN, DEGW), lambda i: (i, 0))
  bs_w = pl.BlockSpec((D, D), lambda i: (0, 0))
  bs_v = pl.BlockSpec((1, D), lambda i: (0, 0))
  return pl.pallas_call(
      functools.partial(_tc_body, relu),
      grid=(N // BN,),
      in_specs=[bs_rows, bs_rows, bs_rows, bs_deg, bs_deg,
                bs_w, bs_w, bs_v, bs_v, bs_v],
      out_specs=bs_rows,
      out_shape=jax.ShapeDtypeStruct((N, D), jnp.float32),
  )(h, a0, a1, d0, d1, ws, wn, b, g, bt)


_sc_agg = _make_sc_agg()
_sc_deg = _make_sc_deg()


def kernel(x, edge_index, W_self0, W_neigh0, b0, W_self1, W_neigh1, b1,
           ln_gamma, ln_beta):
  src = jnp.concatenate([edge_index[0].astype(jnp.int32),
                         jnp.zeros((E_PAD - E,), jnp.int32)])
  dst = jnp.concatenate([edge_index[1].astype(jnp.int32),
                         N + jnp.arange(E_PAD - E, dtype=jnp.int32)
                         % (N_PAD - N)])
  znd = jnp.zeros((N_PAD, D), jnp.float32)
  z16 = jnp.zeros((N_PAD, DEGW), jnp.float32)
  ones = jnp.ones((K, DEGW), jnp.float32)
  b0r, b1r = b0.reshape(1, D), b1.reshape(1, D)
  gr, btr = ln_gamma.reshape(1, D), ln_beta.reshape(1, D)

  degp = _sc_deg(dst, z16, ones)
  agg0 = _sc_agg(x, src, dst, znd)
  h1 = _tc_dense(x, agg0[0], agg0[1], degp[0], degp[1],
                 W_self0, W_neigh0, b0r, gr, btr, True)
  agg1 = _sc_agg(h1, src, dst, znd)
  out = _tc_dense(h1, agg1[0], agg1[1], degp[0], degp[1],
                  W_self1, W_neigh1, b1r, gr, btr, False)
  return out
